# Initial kernel scaffold; baseline (speedup 1.0000x reference)
import functools

import jax
import jax.numpy as jnp
from jax import lax
from jax.experimental import pallas as pl
from jax.experimental.pallas import tpu as pltpu
from jax.experimental.pallas import tpu_sc as plsc

HIDDEN = 64
BATCH = 16384
HIST = 50
TOTAL = BATCH * HIST
NC, NS = 2, 16
NW = NC * NS
PER_W = TOTAL // NW
CHUNK = 512
N_CHUNKS = PER_W // CHUNK

_mesh = plsc.VectorSubcoreMesh(core_axis_name="c", subcore_axis_name="s")


@functools.partial(
    pl.kernel,
    mesh=_mesh,
    out_type=jax.ShapeDtypeStruct((TOTAL, HIDDEN), jnp.float32),
    scratch_types=[
        pltpu.VMEM((CHUNK,), jnp.int32),
        pltpu.VMEM((CHUNK, HIDDEN), jnp.float32),
        pltpu.SemaphoreType.DMA,
    ],
)
def _gather(idx_hbm, table_hbm, out_hbm, idx_v, rows_v, sem):
    wid = lax.axis_index("s") * NC + lax.axis_index("c")
    base = wid * PER_W

    def body(c, carry):
        off = base + c * CHUNK
        pltpu.sync_copy(idx_hbm.at[pl.ds(off, CHUNK)], idx_v)
        pltpu.async_copy(table_hbm.at[idx_v], rows_v, sem).wait()
        pltpu.sync_copy(rows_v, out_hbm.at[pl.ds(off, CHUNK)])
        return carry

    lax.fori_loop(0, N_CHUNKS, body, 0)


def kernel(input_ids, embedding):
    idx = input_ids.reshape(TOTAL)
    out = _gather(idx, embedding)
    return out.reshape(BATCH, HIST, HIDDEN)


# speedup vs baseline: 1.7974x; 1.7974x over previous
import functools

import jax
import jax.numpy as jnp
from jax import lax
from jax.experimental import pallas as pl
from jax.experimental.pallas import tpu as pltpu
from jax.experimental.pallas import tpu_sc as plsc

HIDDEN = 64
BATCH = 16384
HIST = 50
TOTAL = BATCH * HIST
NC, NS = 2, 16
NW = NC * NS
PER_W = TOTAL // NW
CHUNK = 512
N_CHUNKS = PER_W // CHUNK

_mesh = plsc.VectorSubcoreMesh(core_axis_name="c", subcore_axis_name="s")


@functools.partial(
    pl.kernel,
    mesh=_mesh,
    out_type=jax.ShapeDtypeStruct((TOTAL, HIDDEN), jnp.float32),
    scratch_types=[
        pltpu.VMEM((CHUNK,), jnp.int32),
        pltpu.VMEM((CHUNK, HIDDEN), jnp.float32),
        pltpu.SemaphoreType.DMA,
    ],
    compiler_params=pltpu.CompilerParams(use_tc_tiling_on_sc=False),
)
def _gather(idx_hbm, table_hbm, out_hbm, idx_v, rows_v, sem):
    wid = lax.axis_index("s") * NC + lax.axis_index("c")
    base = wid * PER_W

    def body(c, carry):
        off = base + c * CHUNK
        pltpu.sync_copy(idx_hbm.at[pl.ds(off, CHUNK)], idx_v)
        pltpu.async_copy(table_hbm.at[idx_v], rows_v, sem).wait()
        pltpu.sync_copy(rows_v, out_hbm.at[pl.ds(off, CHUNK)])
        return carry

    lax.fori_loop(0, N_CHUNKS, body, 0)


def kernel(input_ids, embedding):
    idx = input_ids.reshape(TOTAL)
    out = _gather(idx, embedding)
    return out.reshape(BATCH, HIST, HIDDEN)


# speedup vs baseline: 1.8721x; 1.0415x over previous
import functools

import jax
import jax.numpy as jnp
from jax import lax
from jax.experimental import pallas as pl
from jax.experimental.pallas import tpu as pltpu
from jax.experimental.pallas import tpu_sc as plsc

HIDDEN = 64
BATCH = 16384
HIST = 50
TOTAL = BATCH * HIST
NC, NS = 2, 16
NW = NC * NS
PER_W = TOTAL // NW
CHUNK = 800
N_CHUNKS = PER_W // CHUNK
N_PAIRS = N_CHUNKS // 2

_mesh = plsc.VectorSubcoreMesh(core_axis_name="c", subcore_axis_name="s")


@functools.partial(
    pl.kernel,
    mesh=_mesh,
    out_type=jax.ShapeDtypeStruct((TOTAL, HIDDEN), jnp.float32),
    scratch_types=[
        pltpu.VMEM((PER_W,), jnp.int32),
        pltpu.VMEM((CHUNK, HIDDEN), jnp.float32),
        pltpu.VMEM((CHUNK, HIDDEN), jnp.float32),
        pltpu.SemaphoreType.DMA,
        pltpu.SemaphoreType.DMA,
        pltpu.SemaphoreType.DMA,
        pltpu.SemaphoreType.DMA,
    ],
    compiler_params=pltpu.CompilerParams(use_tc_tiling_on_sc=False),
)
def _gather(idx_hbm, table_hbm, out_hbm, idx_v, rb0, rb1, g0, g1, s0, s1):
    wid = lax.axis_index("s") * NC + lax.axis_index("c")
    base = wid * PER_W

    pltpu.sync_copy(idx_hbm.at[pl.ds(base, PER_W)], idx_v)

    def g_copy(c, rbuf, sem):
        return pltpu.make_async_copy(
            table_hbm.at[idx_v.at[pl.ds(c * CHUNK, CHUNK)]], rbuf, sem)

    def s_copy(c, rbuf, sem):
        return pltpu.make_async_copy(
            rbuf, out_hbm.at[pl.ds(base + c * CHUNK, CHUNK)], sem)

    g_copy(0, rb0, g0).start()
    g_copy(0, rb0, g0).wait()
    g_copy(1, rb1, g1).start()
    s_copy(0, rb0, s0).start()

    def body(p, carry):
        c0 = 2 * p
        g_copy(c0 - 1, rb1, g1).wait()
        s_copy(c0 - 2, rb0, s0).wait()
        g_copy(c0, rb0, g0).start()
        s_copy(c0 - 1, rb1, s1).start()
        g_copy(c0, rb0, g0).wait()
        s_copy(c0 - 1, rb1, s1).wait()
        g_copy(c0 + 1, rb1, g1).start()
        s_copy(c0, rb0, s0).start()
        return carry

    lax.fori_loop(1, N_PAIRS, body, 0)

    g_copy(N_CHUNKS - 1, rb1, g1).wait()
    s_copy(N_CHUNKS - 1, rb1, s1).start()
    s_copy(N_CHUNKS - 2, rb0, s0).wait()
    s_copy(N_CHUNKS - 1, rb1, s1).wait()


def kernel(input_ids, embedding):
    idx = input_ids.reshape(TOTAL)
    out = _gather(idx, embedding)
    return out.reshape(BATCH, HIST, HIDDEN)
